# baseline (speedup 1.0000x reference)
import functools

import jax
import jax.numpy as jnp
from jax import lax
from jax.experimental import pallas as pl
from jax.experimental.pallas import tpu as pltpu

EPS = 1e-5
NNODES = 50000
NEDGES = 1600000
NGRAPH = 30
NF = 7
EF = 6
H = 64
GF = 64
N_MP = 3

TE = 3200
NB = 256
CHUNK = TE
NBLK = (NNODES + NB - 1) // NB
NPAD = NBLK * NB
EGRID = -(-NEDGES // TE) + 1
EPAD = EGRID * TE
TN = 3584
NGRID = NPAD // TN


def _lrelu(x):
    return jnp.where(x >= 0, x, 0.01 * x)


def _bnvec(stats, n):
    s1 = jnp.sum(stats[0:8], axis=0)
    s2 = jnp.sum(stats[8:16], axis=0)
    m = s1 / n
    v = s2 / n - m * m
    return m.reshape(1, -1), jnp.sqrt(v + EPS).reshape(1, -1)


def _padcols(a, w):
    return jnp.pad(a, ((0, 0), (0, w - a.shape[1])))


def _stats16(a):
    n8 = a.shape[0] // 8
    a3 = a.reshape(n8, 8, a.shape[1])
    return jnp.concatenate([jnp.sum(a3, axis=0), jnp.sum(a3 * a3, axis=0)], axis=0)


def _estats_body(xs_ref, xd_ref, ea_ref, W_ref, b_ref, out_ref):
    i = pl.program_id(0)
    feats = jnp.concatenate([xs_ref[...], xd_ref[...], ea_ref[...]], axis=1)
    z = jnp.dot(feats, W_ref[...], preferred_element_type=jnp.float32) + b_ref[...]
    a = _lrelu(z)
    eidx = i * TE + lax.broadcasted_iota(jnp.int32, (TE, 1), 0)
    a = jnp.where(eidx < NEDGES, a, 0.0)

    @pl.when(i == 0)
    def _():
        out_ref[...] = jnp.zeros_like(out_ref)

    out_ref[...] += _stats16(a)


def _edge_stats(xs, xd, ea, W, b):
    return pl.pallas_call(
        _estats_body,
        grid=(EGRID,),
        in_specs=[
            pl.BlockSpec((TE, 8), lambda i: (i, 0)),
            pl.BlockSpec((TE, 8), lambda i: (i, 0)),
            pl.BlockSpec((TE, 8), lambda i: (i, 0)),
            pl.BlockSpec((24, H), lambda i: (0, 0)),
            pl.BlockSpec((1, H), lambda i: (0, 0)),
        ],
        out_specs=pl.BlockSpec((16, H), lambda i: (0, 0)),
        out_shape=jax.ShapeDtypeStruct((16, H), jnp.float32),
    )(xs, xd, ea, W, b)


def _etrans_body(xs_ref, xd_ref, ea_ref, W1_ref, b1_ref, m_ref, s_ref,
                 g_ref, bb_ref, W2_ref, b2_ref, Wh_ref, bh_ref, ean_ref, st_ref):
    i = pl.program_id(0)
    xs = xs_ref[...]
    feats = jnp.concatenate([xs, xd_ref[...], ea_ref[...]], axis=1)
    z = jnp.dot(feats, W1_ref[...], preferred_element_type=jnp.float32) + b1_ref[...]
    a1 = (_lrelu(z) - m_ref[...]) / s_ref[...] * g_ref[...] + bb_ref[...]
    ean = jnp.dot(a1, W2_ref[...], preferred_element_type=jnp.float32) + b2_ref[...]
    ean_ref[...] = ean
    zh = jnp.dot(jnp.concatenate([xs, ean], axis=1), Wh_ref[...],
                 preferred_element_type=jnp.float32) + bh_ref[...]
    ah = _lrelu(zh)
    eidx = i * TE + lax.broadcasted_iota(jnp.int32, (TE, 1), 0)
    ah = jnp.where(eidx < NEDGES, ah, 0.0)

    @pl.when(i == 0)
    def _():
        st_ref[...] = jnp.zeros_like(st_ref)

    st_ref[...] += _stats16(ah)


def _edge_transform(xs, xd, ea, W1, b1, m, sv, g, bb, W2, b2, Wh, bh):
    return pl.pallas_call(
        _etrans_body,
        grid=(EGRID,),
        in_specs=[
            pl.BlockSpec((TE, 8), lambda i: (i, 0)),
            pl.BlockSpec((TE, 8), lambda i: (i, 0)),
            pl.BlockSpec((TE, 8), lambda i: (i, 0)),
            pl.BlockSpec((24, H), lambda i: (0, 0)),
            pl.BlockSpec((1, H), lambda i: (0, 0)),
            pl.BlockSpec((1, H), lambda i: (0, 0)),
            pl.BlockSpec((1, H), lambda i: (0, 0)),
            pl.BlockSpec((1, H), lambda i: (0, 0)),
            pl.BlockSpec((1, H), lambda i: (0, 0)),
            pl.BlockSpec((H, 8), lambda i: (0, 0)),
            pl.BlockSpec((1, 8), lambda i: (0, 0)),
            pl.BlockSpec((16, H), lambda i: (0, 0)),
            pl.BlockSpec((1, H), lambda i: (0, 0)),
        ],
        out_specs=[
            pl.BlockSpec((TE, 8), lambda i: (i, 0)),
            pl.BlockSpec((16, H), lambda i: (0, 0)),
        ],
        out_shape=[
            jax.ShapeDtypeStruct((EPAD, 8), jnp.float32),
            jax.ShapeDtypeStruct((16, H), jnp.float32),
        ],
    )(xs, xd, ea, W1, b1, m, sv, g, bb, W2, b2, Wh, bh)


def _agg_body(ptr_ref, xs_hbm, ean_hbm, col_hbm, Wh_ref, bh_ref, m_ref, s_ref,
              g_ref, bb_ref, Wp_ref, bp_ref, batch_ref, agg_ref, pooled_ref,
              xs_v, ean_v, col_v, s0_, s1_, s2_):
    b = pl.program_id(0)
    start = ptr_ref[b]
    end = ptr_ref[b + 1]
    s0 = (start // 8) * 8
    trips = (end - s0 + CHUNK - 1) // CHUNK

    def body(t, acc):
        off = pl.multiple_of(s0 + t * CHUNK, 8)
        c1 = pltpu.make_async_copy(xs_hbm.at[pl.ds(off, CHUNK), :], xs_v, s0_)
        c2 = pltpu.make_async_copy(ean_hbm.at[pl.ds(off, CHUNK), :], ean_v, s1_)
        c3 = pltpu.make_async_copy(col_hbm.at[pl.ds(off, CHUNK), :], col_v, s2_)
        c1.start(); c2.start(); c3.start()
        c1.wait(); c2.wait(); c3.wait()
        zh = jnp.dot(jnp.concatenate([xs_v[...], ean_v[...]], axis=1), Wh_ref[...],
                     preferred_element_type=jnp.float32) + bh_ref[...]
        ah = (_lrelu(zh) - m_ref[...]) / s_ref[...] * g_ref[...] + bb_ref[...]
        h = jnp.dot(ah, Wp_ref[...], preferred_element_type=jnp.float32) + bp_ref[...]
        eidx = off + lax.broadcasted_iota(jnp.int32, (CHUNK, 1), 0)
        h = jnp.where((eidx >= start) & (eidx < end), h, 0.0)
        lc = jnp.clip(col_v[...] - b * NB, 0, NB - 1)
        oh = (lc == lax.broadcasted_iota(jnp.int32, (1, NB), 1)).astype(jnp.float32)
        return acc + lax.dot_general(oh, h, (((0,), (0,)), ((), ())),
                                     precision=lax.Precision.HIGHEST,
                                     preferred_element_type=jnp.float32)

    acc = lax.fori_loop(0, trips, body, jnp.zeros((NB, H), jnp.float32))
    agg_ref[...] = acc
    bo = (batch_ref[...] == lax.broadcasted_iota(jnp.int32, (1, 32), 1)
          ).astype(jnp.float32)
    pc = lax.dot_general(bo, acc, (((0,), (0,)), ((), ())),
                         precision=lax.Precision.HIGHEST,
                         preferred_element_type=jnp.float32)

    @pl.when(b == 0)
    def _():
        pooled_ref[...] = jnp.zeros_like(pooled_ref)

    pooled_ref[...] += pc


def _aggregate(block_ptr, xs, ean, colp, Wh, bh, m, sv, g, bb, Wp, bp, batch_pad):
    grid_spec = pltpu.PrefetchScalarGridSpec(
        num_scalar_prefetch=1,
        grid=(NBLK,),
        in_specs=[
            pl.BlockSpec(memory_space=pl.ANY),
            pl.BlockSpec(memory_space=pl.ANY),
            pl.BlockSpec(memory_space=pl.ANY),
            pl.BlockSpec((16, H), lambda b, p: (0, 0)),
            pl.BlockSpec((1, H), lambda b, p: (0, 0)),
            pl.BlockSpec((1, H), lambda b, p: (0, 0)),
            pl.BlockSpec((1, H), lambda b, p: (0, 0)),
            pl.BlockSpec((1, H), lambda b, p: (0, 0)),
            pl.BlockSpec((1, H), lambda b, p: (0, 0)),
            pl.BlockSpec((H, H), lambda b, p: (0, 0)),
            pl.BlockSpec((1, H), lambda b, p: (0, 0)),
            pl.BlockSpec((NB, 1), lambda b, p: (b, 0)),
        ],
        out_specs=[
            pl.BlockSpec((NB, H), lambda b, p: (b, 0)),
            pl.BlockSpec((32, H), lambda b, p: (0, 0)),
        ],
        scratch_shapes=[
            pltpu.VMEM((CHUNK, 8), jnp.float32),
            pltpu.VMEM((CHUNK, 8), jnp.float32),
            pltpu.VMEM((CHUNK, 1), jnp.int32),
            pltpu.SemaphoreType.DMA,
            pltpu.SemaphoreType.DMA,
            pltpu.SemaphoreType.DMA,
        ],
    )
    return pl.pallas_call(
        _agg_body,
        grid_spec=grid_spec,
        out_shape=[
            jax.ShapeDtypeStruct((NPAD, H), jnp.float32),
            jax.ShapeDtypeStruct((32, H), jnp.float32),
        ],
    )(block_ptr, xs, ean, colp, Wh, bh, m, sv, g, bb, Wp, bp, batch_pad)


def _nstats_body(x_ref, agg_ref, W_ref, b_ref, out_ref):
    i = pl.program_id(0)
    feats = jnp.concatenate([x_ref[...], agg_ref[...]], axis=1)
    z = jnp.dot(feats, W_ref[...], preferred_element_type=jnp.float32) + b_ref[...]
    a = _lrelu(z)
    nidx = i * TN + lax.broadcasted_iota(jnp.int32, (TN, 1), 0)
    a = jnp.where(nidx < NNODES, a, 0.0)

    @pl.when(i == 0)
    def _():
        out_ref[...] = jnp.zeros_like(out_ref)

    out_ref[...] += _stats16(a)


def _node_stats(xp, agg, W, b):
    return pl.pallas_call(
        _nstats_body,
        grid=(NGRID,),
        in_specs=[
            pl.BlockSpec((TN, 8), lambda i: (i, 0)),
            pl.BlockSpec((TN, H), lambda i: (i, 0)),
            pl.BlockSpec((72, H), lambda i: (0, 0)),
            pl.BlockSpec((1, H), lambda i: (0, 0)),
        ],
        out_specs=pl.BlockSpec((16, H), lambda i: (0, 0)),
        out_shape=jax.ShapeDtypeStruct((16, H), jnp.float32),
    )(xp, agg, W, b)


def _ntrans_body(x_ref, agg_ref, W_ref, b_ref, m_ref, s_ref, g_ref, bb_ref,
                 Wp_ref, bp_ref, batch_ref, xn_ref, px_ref):
    i = pl.program_id(0)
    feats = jnp.concatenate([x_ref[...], agg_ref[...]], axis=1)
    z = jnp.dot(feats, W_ref[...], preferred_element_type=jnp.float32) + b_ref[...]
    an = (_lrelu(z) - m_ref[...]) / s_ref[...] * g_ref[...] + bb_ref[...]
    xn = jnp.dot(an, Wp_ref[...], preferred_element_type=jnp.float32) + bp_ref[...]
    xn_ref[...] = xn
    bo = (batch_ref[...] == lax.broadcasted_iota(jnp.int32, (1, 32), 1)
          ).astype(jnp.float32)
    pc = lax.dot_general(bo, xn, (((0,), (0,)), ((), ())),
                         precision=lax.Precision.HIGHEST,
                         preferred_element_type=jnp.float32)

    @pl.when(i == 0)
    def _():
        px_ref[...] = jnp.zeros_like(px_ref)

    px_ref[...] += pc


def _node_transform(xp, agg, W, b, m, sv, g, bb, Wp, bp, batch_pad):
    return pl.pallas_call(
        _ntrans_body,
        grid=(NGRID,),
        in_specs=[
            pl.BlockSpec((TN, 8), lambda i: (i, 0)),
            pl.BlockSpec((TN, H), lambda i: (i, 0)),
            pl.BlockSpec((72, H), lambda i: (0, 0)),
            pl.BlockSpec((1, H), lambda i: (0, 0)),
            pl.BlockSpec((1, H), lambda i: (0, 0)),
            pl.BlockSpec((1, H), lambda i: (0, 0)),
            pl.BlockSpec((1, H), lambda i: (0, 0)),
            pl.BlockSpec((1, H), lambda i: (0, 0)),
            pl.BlockSpec((H, 8), lambda i: (0, 0)),
            pl.BlockSpec((1, 8), lambda i: (0, 0)),
            pl.BlockSpec((TN, 1), lambda i: (i, 0)),
        ],
        out_specs=[
            pl.BlockSpec((TN, 8), lambda i: (i, 0)),
            pl.BlockSpec((32, 8), lambda i: (0, 0)),
        ],
        out_shape=[
            jax.ShapeDtypeStruct((NPAD, 8), jnp.float32),
            jax.ShapeDtypeStruct((32, 8), jnp.float32),
        ],
    )(xp, agg, W, b, m, sv, g, bb, Wp, bp, batch_pad)


def _global_body(px_ref, pg_ref, Wa_ref, ba_ref, ga_ref, bba_ref,
                 Wb_ref, bb_ref, gb_ref, bbb_ref,
                 Wc_ref, bc_ref, gc_ref, bbc_ref,
                 Wd_ref, bd_ref, u_ref):
    p = jnp.concatenate([px_ref[...], pg_ref[...]], axis=1)
    valid = lax.broadcasted_iota(jnp.int32, (32, 1), 0) < NGRAPH

    def layer(p, W, b, g, bb):
        z = jnp.dot(p, W[...], preferred_element_type=jnp.float32) + b[...]
        a = _lrelu(z)
        am = jnp.where(valid, a, 0.0)
        m = jnp.sum(am, axis=0, keepdims=True) / NGRAPH
        v = jnp.sum(am * am, axis=0, keepdims=True) / NGRAPH - m * m
        return (a - m) / jnp.sqrt(v + EPS) * g[...] + bb[...]

    p = layer(p, Wa_ref, ba_ref, ga_ref, bba_ref)
    p = layer(p, Wb_ref, bb_ref, gb_ref, bbb_ref)
    p = layer(p, Wc_ref, bc_ref, gc_ref, bbc_ref)
    u_ref[...] = jnp.dot(p, Wd_ref[...], preferred_element_type=jnp.float32) + bd_ref[...]


def _global_mlp(px, pg, Wa, ba, ga, bba, Wb, bb, gb, bbb, Wc, bc, gc, bbc, Wd, bd):
    args = (px, pg, Wa, ba, ga, bba, Wb, bb, gb, bbb, Wc, bc, gc, bbc, Wd, bd)
    return pl.pallas_call(
        _global_body,
        out_shape=jax.ShapeDtypeStruct((32, GF), jnp.float32),
    )(*args)


def kernel(x, edge_index, edge_attr, batch, params):
    p = params
    row = edge_index[0]
    col = edge_index[1]

    perm = jnp.argsort(col)
    rowp = row[perm]
    colp = col[perm]
    eap = jnp.pad(edge_attr, ((0, EPAD - NEDGES), (0, 2)))[
        jnp.concatenate([perm, jnp.arange(NEDGES, EPAD, dtype=perm.dtype)])]
    colp2 = jnp.pad(colp, (0, EPAD - NEDGES)).reshape(EPAD, 1)
    block_ptr = jnp.searchsorted(
        colp, jnp.arange(NBLK + 1, dtype=jnp.int32) * NB).astype(jnp.int32)

    W1e = jnp.concatenate([
        _padcols(p['We1'][:NF].T, 8).T, _padcols(p['We1'][NF:2 * NF].T, 8).T,
        _padcols(p['We1'][2 * NF:].T, 8).T], axis=0)
    b1e = p['be1'].reshape(1, H)
    Wn1 = jnp.concatenate([
        _padcols(p['Wn1'][:NF].T, 8).T, _padcols(p['Wn1'][NF:].T, 8).T], axis=0)
    bn1 = p['bn1'].reshape(1, H)
    Wg1 = jnp.concatenate([
        _padcols(p['Wg1'][:NF].T, 8).T, _padcols(p['Wg1'][NF:].T, 8).T], axis=0)
    bg1 = p['bg1'].reshape(1, H)
    Wn2 = jnp.concatenate([p['Wn2'][:NF], jnp.zeros((1, H), jnp.float32),
                           p['Wn2'][NF:]], axis=0)
    bn2 = p['bn2'].reshape(1, H)

    xp = jnp.zeros((NPAD, 8), jnp.float32).at[:NNODES, :NF].set(x)
    batch_pad = jnp.concatenate(
        [batch, jnp.full((NPAD - NNODES,), NGRAPH, jnp.int32)]).reshape(NPAD, 1)

    u = None
    for it in range(N_MP):
        xs = jnp.take(xp, rowp, axis=0)
        xs = jnp.pad(xs, ((0, EPAD - NEDGES), (0, 0)))
        xd = jnp.take(xp, colp, axis=0)
        xd = jnp.pad(xd, ((0, EPAD - NEDGES), (0, 0)))

        st1 = _edge_stats(xs, xd, eap, W1e, b1e)
        m1, s1v = _bnvec(st1, NEDGES)
        We2 = _padcols(p['We2'], 8)
        be2 = _padcols(p['be2'].reshape(1, EF), 8)

        ean, sth = _edge_transform(xs, xd, eap, W1e, b1e, m1, s1v,
                                   p['ge1'].reshape(1, H), p['bbe1'].reshape(1, H),
                                   We2, be2, Wn1, bn1)
        mh, shv = _bnvec(sth, NEDGES)

        agg, _ = _aggregate(block_ptr, xs, ean, colp2, Wn1, bn1, mh, shv,
                            p['gn1'].reshape(1, H), p['bbn1'].reshape(1, H),
                            p['Wn1b'], p['bn1b'].reshape(1, H), batch_pad)

        stn = _node_stats(xp, agg, Wn2, bn2)
        mn, snv = _bnvec(stn, NNODES)
        Wn2b = _padcols(p['Wn2b'], 8)
        bn2b = _padcols(p['bn2b'].reshape(1, NF), 8)

        xp, px = _node_transform(xp, agg, Wn2, bn2, mn, snv,
                                 p['gn2'].reshape(1, H), p['bbn2'].reshape(1, H),
                                 Wn2b, bn2b, batch_pad)

        if it == N_MP - 1:
            xs2 = jnp.take(xp, rowp, axis=0)
            xs2 = jnp.pad(xs2, ((0, EPAD - NEDGES), (0, 0)))
            stg = _edge_stats(xs2, jnp.zeros_like(xs2), ean,
                              jnp.concatenate([Wg1[:8], jnp.zeros((8, H), jnp.float32),
                                               Wg1[8:]], axis=0), bg1)
            mg, sgv = _bnvec(stg, NEDGES)
            _, pg = _aggregate(block_ptr, xs2, ean, colp2, Wg1, bg1, mg, sgv,
                               p['gg1'].reshape(1, H), p['bbg1'].reshape(1, H),
                               p['Wg1b'], p['bg1b'].reshape(1, H), batch_pad)
            Wg2a = jnp.concatenate(
                [p['Wg2a'][:NF], jnp.zeros((1, H), jnp.float32), p['Wg2a'][NF:]],
                axis=0)
            u = _global_mlp(
                px, pg,
                Wg2a, p['bg2a'].reshape(1, H), p['gg2a'].reshape(1, H),
                p['bbg2a'].reshape(1, H),
                p['Wg2b'], p['bg2b'].reshape(1, H), p['gg2b'].reshape(1, H),
                p['bbg2b'].reshape(1, H),
                p['Wg2c'], p['bg2c'].reshape(1, H), p['gg2c'].reshape(1, H),
                p['bbg2c'].reshape(1, H),
                p['Wg2d'], p['bg2d'].reshape(1, GF))[:NGRAPH]
        eap = ean

    x_out = xp[:NNODES, :NF]
    ea_out = jnp.zeros((NEDGES, EF), jnp.float32).at[perm].set(eap[:NEDGES, :EF])
    return (x_out, ea_out, u)


# speedup vs baseline: 1.0003x; 1.0003x over previous
import jax
import jax.numpy as jnp
from jax import lax
from jax.experimental import pallas as pl
from jax.experimental.pallas import tpu as pltpu

EPS = 1e-5
NNODES = 50000
NEDGES = 1600000
NGRAPH = 30
NF = 7
EF = 6
H = 64
GF = 64
N_MP = 3

TE = 3200
NB = 256
CHUNK = TE
NBLK = (NNODES + NB - 1) // NB
NPAD = NBLK * NB
EGRID = -(-NEDGES // TE) + 1
EPAD = EGRID * TE
TN = 3584
NGRID = NPAD // TN


def _lrelu(x):
    return jnp.where(x >= 0, x, 0.01 * x)


def _bnvec(stats, n):
    s1 = jnp.sum(stats[0:8], axis=0)
    s2 = jnp.sum(stats[8:16], axis=0)
    m = s1 / n
    v = s2 / n - m * m
    return m.reshape(1, -1), jnp.sqrt(v + EPS).reshape(1, -1)


def _padcols(a, w):
    return jnp.pad(a, ((0, 0), (0, w - a.shape[1])))


def _stats16(a):
    n8 = a.shape[0] // 8
    a3 = a.reshape(n8, 8, a.shape[1])
    return jnp.concatenate([jnp.sum(a3, axis=0), jnp.sum(a3 * a3, axis=0)], axis=0)


def _estats_body(xs_ref, xd_ref, ea_ref, W_ref, b_ref, out_ref):
    i = pl.program_id(0)
    feats = jnp.concatenate([xs_ref[...], xd_ref[...], ea_ref[...]], axis=1)
    z = jnp.dot(feats, W_ref[...], preferred_element_type=jnp.float32) + b_ref[...]
    a = _lrelu(z)
    eidx = i * TE + lax.broadcasted_iota(jnp.int32, (TE, 1), 0)
    a = jnp.where(eidx < NEDGES, a, 0.0)

    @pl.when(i == 0)
    def _():
        out_ref[...] = jnp.zeros_like(out_ref)

    out_ref[...] += _stats16(a)


def _edge_stats(xs, xd, ea, W, b):
    return pl.pallas_call(
        _estats_body,
        grid=(EGRID,),
        in_specs=[
            pl.BlockSpec((TE, 8), lambda i: (i, 0)),
            pl.BlockSpec((TE, 8), lambda i: (i, 0)),
            pl.BlockSpec((TE, 8), lambda i: (i, 0)),
            pl.BlockSpec((24, H), lambda i: (0, 0)),
            pl.BlockSpec((1, H), lambda i: (0, 0)),
        ],
        out_specs=pl.BlockSpec((16, H), lambda i: (0, 0)),
        out_shape=jax.ShapeDtypeStruct((16, H), jnp.float32),
    )(xs, xd, ea, W, b)


def _etrans_body(xs_ref, xd_ref, ea_ref, W1_ref, b1_ref, m_ref, s_ref,
                 g_ref, bb_ref, W2_ref, b2_ref, Wh_ref, bh_ref, ean_ref, st_ref):
    i = pl.program_id(0)
    xs = xs_ref[...]
    feats = jnp.concatenate([xs, xd_ref[...], ea_ref[...]], axis=1)
    z = jnp.dot(feats, W1_ref[...], preferred_element_type=jnp.float32) + b1_ref[...]
    a1 = (_lrelu(z) - m_ref[...]) / s_ref[...] * g_ref[...] + bb_ref[...]
    ean = jnp.dot(a1, W2_ref[...], preferred_element_type=jnp.float32) + b2_ref[...]
    ean_ref[...] = ean
    zh = jnp.dot(jnp.concatenate([xs, ean], axis=1), Wh_ref[...],
                 preferred_element_type=jnp.float32) + bh_ref[...]
    ah = _lrelu(zh)
    eidx = i * TE + lax.broadcasted_iota(jnp.int32, (TE, 1), 0)
    ah = jnp.where(eidx < NEDGES, ah, 0.0)

    @pl.when(i == 0)
    def _():
        st_ref[...] = jnp.zeros_like(st_ref)

    st_ref[...] += _stats16(ah)


def _edge_transform(xs, xd, ea, W1, b1, m, sv, g, bb, W2, b2, Wh, bh):
    return pl.pallas_call(
        _etrans_body,
        grid=(EGRID,),
        in_specs=[
            pl.BlockSpec((TE, 8), lambda i: (i, 0)),
            pl.BlockSpec((TE, 8), lambda i: (i, 0)),
            pl.BlockSpec((TE, 8), lambda i: (i, 0)),
            pl.BlockSpec((24, H), lambda i: (0, 0)),
            pl.BlockSpec((1, H), lambda i: (0, 0)),
            pl.BlockSpec((1, H), lambda i: (0, 0)),
            pl.BlockSpec((1, H), lambda i: (0, 0)),
            pl.BlockSpec((1, H), lambda i: (0, 0)),
            pl.BlockSpec((1, H), lambda i: (0, 0)),
            pl.BlockSpec((H, 8), lambda i: (0, 0)),
            pl.BlockSpec((1, 8), lambda i: (0, 0)),
            pl.BlockSpec((16, H), lambda i: (0, 0)),
            pl.BlockSpec((1, H), lambda i: (0, 0)),
        ],
        out_specs=[
            pl.BlockSpec((TE, 8), lambda i: (i, 0)),
            pl.BlockSpec((16, H), lambda i: (0, 0)),
        ],
        out_shape=[
            jax.ShapeDtypeStruct((EPAD, 8), jnp.float32),
            jax.ShapeDtypeStruct((16, H), jnp.float32),
        ],
    )(xs, xd, ea, W1, b1, m, sv, g, bb, W2, b2, Wh, bh)


def _agg_body(ptr_ref, xs_hbm, ean_hbm, col_hbm, Wh_ref, bh_ref, m_ref, s_ref,
              g_ref, bb_ref, Wp_ref, bp_ref, batch_ref, agg_ref, pooled_ref,
              xs_v, ean_v, col_v, s0_, s1_, s2_):
    b = pl.program_id(0)
    start = ptr_ref[b]
    end = ptr_ref[b + 1]
    s0 = (start // 8) * 8
    trips = (end - s0 + CHUNK - 1) // CHUNK

    def body(t, acc):
        off = pl.multiple_of(s0 + t * CHUNK, 8)
        c1 = pltpu.make_async_copy(xs_hbm.at[pl.ds(off, CHUNK), :], xs_v, s0_)
        c2 = pltpu.make_async_copy(ean_hbm.at[pl.ds(off, CHUNK), :], ean_v, s1_)
        c3 = pltpu.make_async_copy(col_hbm.at[pl.ds(off, CHUNK), :], col_v, s2_)
        c1.start(); c2.start(); c3.start()
        c1.wait(); c2.wait(); c3.wait()
        zh = jnp.dot(jnp.concatenate([xs_v[...], ean_v[...]], axis=1), Wh_ref[...],
                     preferred_element_type=jnp.float32) + bh_ref[...]
        ah = (_lrelu(zh) - m_ref[...]) / s_ref[...] * g_ref[...] + bb_ref[...]
        h = jnp.dot(ah, Wp_ref[...], preferred_element_type=jnp.float32) + bp_ref[...]
        eidx = off + lax.broadcasted_iota(jnp.int32, (CHUNK, 1), 0)
        h = jnp.where((eidx >= start) & (eidx < end), h, 0.0)
        lc = jnp.clip(col_v[...] - b * NB, 0, NB - 1)
        oh = (lc == lax.broadcasted_iota(jnp.int32, (1, NB), 1)).astype(jnp.float32)
        return acc + lax.dot_general(oh, h, (((0,), (0,)), ((), ())),
                                     precision=lax.Precision.HIGHEST,
                                     preferred_element_type=jnp.float32)

    acc = lax.fori_loop(0, trips, body, jnp.zeros((NB, H), jnp.float32))
    agg_ref[...] = acc
    bo = (batch_ref[...] == lax.broadcasted_iota(jnp.int32, (1, 32), 1)
          ).astype(jnp.float32)
    pc = lax.dot_general(bo, acc, (((0,), (0,)), ((), ())),
                         precision=lax.Precision.HIGHEST,
                         preferred_element_type=jnp.float32)

    @pl.when(b == 0)
    def _():
        pooled_ref[...] = jnp.zeros_like(pooled_ref)

    pooled_ref[...] += pc


def _aggregate(block_ptr, xs, ean, colp, Wh, bh, m, sv, g, bb, Wp, bp, batch_pad):
    grid_spec = pltpu.PrefetchScalarGridSpec(
        num_scalar_prefetch=1,
        grid=(NBLK,),
        in_specs=[
            pl.BlockSpec(memory_space=pl.ANY),
            pl.BlockSpec(memory_space=pl.ANY),
            pl.BlockSpec(memory_space=pl.ANY),
            pl.BlockSpec((16, H), lambda b, p: (0, 0)),
            pl.BlockSpec((1, H), lambda b, p: (0, 0)),
            pl.BlockSpec((1, H), lambda b, p: (0, 0)),
            pl.BlockSpec((1, H), lambda b, p: (0, 0)),
            pl.BlockSpec((1, H), lambda b, p: (0, 0)),
            pl.BlockSpec((1, H), lambda b, p: (0, 0)),
            pl.BlockSpec((H, H), lambda b, p: (0, 0)),
            pl.BlockSpec((1, H), lambda b, p: (0, 0)),
            pl.BlockSpec((NB, 1), lambda b, p: (b, 0)),
        ],
        out_specs=[
            pl.BlockSpec((NB, H), lambda b, p: (b, 0)),
            pl.BlockSpec((32, H), lambda b, p: (0, 0)),
        ],
        scratch_shapes=[
            pltpu.VMEM((CHUNK, 8), jnp.float32),
            pltpu.VMEM((CHUNK, 8), jnp.float32),
            pltpu.VMEM((CHUNK, 1), jnp.int32),
            pltpu.SemaphoreType.DMA,
            pltpu.SemaphoreType.DMA,
            pltpu.SemaphoreType.DMA,
        ],
    )
    return pl.pallas_call(
        _agg_body,
        grid_spec=grid_spec,
        out_shape=[
            jax.ShapeDtypeStruct((NPAD, H), jnp.float32),
            jax.ShapeDtypeStruct((32, H), jnp.float32),
        ],
    )(block_ptr, xs, ean, colp, Wh, bh, m, sv, g, bb, Wp, bp, batch_pad)


def _nstats_body(x_ref, agg_ref, W_ref, b_ref, out_ref):
    i = pl.program_id(0)
    feats = jnp.concatenate([x_ref[...], agg_ref[...]], axis=1)
    z = jnp.dot(feats, W_ref[...], preferred_element_type=jnp.float32) + b_ref[...]
    a = _lrelu(z)
    nidx = i * TN + lax.broadcasted_iota(jnp.int32, (TN, 1), 0)
    a = jnp.where(nidx < NNODES, a, 0.0)

    @pl.when(i == 0)
    def _():
        out_ref[...] = jnp.zeros_like(out_ref)

    out_ref[...] += _stats16(a)


def _node_stats(xp, agg, W, b):
    return pl.pallas_call(
        _nstats_body,
        grid=(NGRID,),
        in_specs=[
            pl.BlockSpec((TN, 8), lambda i: (i, 0)),
            pl.BlockSpec((TN, H), lambda i: (i, 0)),
            pl.BlockSpec((72, H), lambda i: (0, 0)),
            pl.BlockSpec((1, H), lambda i: (0, 0)),
        ],
        out_specs=pl.BlockSpec((16, H), lambda i: (0, 0)),
        out_shape=jax.ShapeDtypeStruct((16, H), jnp.float32),
    )(xp, agg, W, b)


def _ntrans_body(x_ref, agg_ref, W_ref, b_ref, m_ref, s_ref, g_ref, bb_ref,
                 Wp_ref, bp_ref, batch_ref, xn_ref, px_ref):
    i = pl.program_id(0)
    feats = jnp.concatenate([x_ref[...], agg_ref[...]], axis=1)
    z = jnp.dot(feats, W_ref[...], preferred_element_type=jnp.float32) + b_ref[...]
    an = (_lrelu(z) - m_ref[...]) / s_ref[...] * g_ref[...] + bb_ref[...]
    xn = jnp.dot(an, Wp_ref[...], preferred_element_type=jnp.float32) + bp_ref[...]
    xn_ref[...] = xn
    bo = (batch_ref[...] == lax.broadcasted_iota(jnp.int32, (1, 32), 1)
          ).astype(jnp.float32)
    pc = lax.dot_general(bo, xn, (((0,), (0,)), ((), ())),
                         precision=lax.Precision.HIGHEST,
                         preferred_element_type=jnp.float32)

    @pl.when(i == 0)
    def _():
        px_ref[...] = jnp.zeros_like(px_ref)

    px_ref[...] += pc


def _node_transform(xp, agg, W, b, m, sv, g, bb, Wp, bp, batch_pad):
    return pl.pallas_call(
        _ntrans_body,
        grid=(NGRID,),
        in_specs=[
            pl.BlockSpec((TN, 8), lambda i: (i, 0)),
            pl.BlockSpec((TN, H), lambda i: (i, 0)),
            pl.BlockSpec((72, H), lambda i: (0, 0)),
            pl.BlockSpec((1, H), lambda i: (0, 0)),
            pl.BlockSpec((1, H), lambda i: (0, 0)),
            pl.BlockSpec((1, H), lambda i: (0, 0)),
            pl.BlockSpec((1, H), lambda i: (0, 0)),
            pl.BlockSpec((1, H), lambda i: (0, 0)),
            pl.BlockSpec((H, 8), lambda i: (0, 0)),
            pl.BlockSpec((1, 8), lambda i: (0, 0)),
            pl.BlockSpec((TN, 1), lambda i: (i, 0)),
        ],
        out_specs=[
            pl.BlockSpec((TN, 8), lambda i: (i, 0)),
            pl.BlockSpec((32, 8), lambda i: (0, 0)),
        ],
        out_shape=[
            jax.ShapeDtypeStruct((NPAD, 8), jnp.float32),
            jax.ShapeDtypeStruct((32, 8), jnp.float32),
        ],
    )(xp, agg, W, b, m, sv, g, bb, Wp, bp, batch_pad)


def _global_body(px_ref, pg_ref, Wa_ref, ba_ref, ga_ref, bba_ref,
                 Wb_ref, bb_ref, gb_ref, bbb_ref,
                 Wc_ref, bc_ref, gc_ref, bbc_ref,
                 Wd_ref, bd_ref, u_ref):
    p = jnp.concatenate([px_ref[...], pg_ref[...]], axis=1)
    valid = lax.broadcasted_iota(jnp.int32, (32, 1), 0) < NGRAPH

    def layer(p, W, b, g, bb):
        z = jnp.dot(p, W[...], preferred_element_type=jnp.float32) + b[...]
        a = _lrelu(z)
        am = jnp.where(valid, a, 0.0)
        m = jnp.sum(am, axis=0, keepdims=True) / NGRAPH
        v = jnp.sum(am * am, axis=0, keepdims=True) / NGRAPH - m * m
        return (a - m) / jnp.sqrt(v + EPS) * g[...] + bb[...]

    p = layer(p, Wa_ref, ba_ref, ga_ref, bba_ref)
    p = layer(p, Wb_ref, bb_ref, gb_ref, bbb_ref)
    p = layer(p, Wc_ref, bc_ref, gc_ref, bbc_ref)
    u_ref[...] = jnp.dot(p, Wd_ref[...], preferred_element_type=jnp.float32) + bd_ref[...]


def _global_mlp(px, pg, Wa, ba, ga, bba, Wb, bb, gb, bbb, Wc, bc, gc, bbc, Wd, bd):
    args = (px, pg, Wa, ba, ga, bba, Wb, bb, gb, bbb, Wc, bc, gc, bbc, Wd, bd)
    return pl.pallas_call(
        _global_body,
        out_shape=jax.ShapeDtypeStruct((32, GF), jnp.float32),
    )(*args)


def kernel(x, edge_index, edge_attr, batch, params):
    p = params
    row = edge_index[0]
    col = edge_index[1]

    perm = jnp.argsort(col)
    rowp = row[perm]
    colp = col[perm]
    eap = jnp.pad(edge_attr, ((0, EPAD - NEDGES), (0, 2)))[
        jnp.concatenate([perm, jnp.arange(NEDGES, EPAD, dtype=perm.dtype)])]
    colp2 = jnp.pad(colp, (0, EPAD - NEDGES)).reshape(EPAD, 1)
    block_ptr = jnp.searchsorted(
        colp, jnp.arange(NBLK + 1, dtype=jnp.int32) * NB).astype(jnp.int32)

    W1e = jnp.concatenate([
        _padcols(p['We1'][:NF].T, 8).T, _padcols(p['We1'][NF:2 * NF].T, 8).T,
        _padcols(p['We1'][2 * NF:].T, 8).T], axis=0)
    b1e = p['be1'].reshape(1, H)
    Wn1 = jnp.concatenate([
        _padcols(p['Wn1'][:NF].T, 8).T, _padcols(p['Wn1'][NF:].T, 8).T], axis=0)
    bn1 = p['bn1'].reshape(1, H)
    Wg1 = jnp.concatenate([
        _padcols(p['Wg1'][:NF].T, 8).T, _padcols(p['Wg1'][NF:].T, 8).T], axis=0)
    bg1 = p['bg1'].reshape(1, H)
    Wn2 = jnp.concatenate([p['Wn2'][:NF], jnp.zeros((1, H), jnp.float32),
                           p['Wn2'][NF:]], axis=0)
    bn2 = p['bn2'].reshape(1, H)

    xp = jnp.zeros((NPAD, 8), jnp.float32).at[:NNODES, :NF].set(x)
    batch_pad = jnp.concatenate(
        [batch, jnp.full((NPAD - NNODES,), NGRAPH, jnp.int32)]).reshape(NPAD, 1)

    u = None
    for it in range(N_MP):
        xs = jnp.take(xp, rowp, axis=0)
        xs = jnp.pad(xs, ((0, EPAD - NEDGES), (0, 0)))
        xd = jnp.take(xp, colp, axis=0)
        xd = jnp.pad(xd, ((0, EPAD - NEDGES), (0, 0)))

        st1 = _edge_stats(xs, xd, eap, W1e, b1e)
        m1, s1v = _bnvec(st1, NEDGES)
        We2 = _padcols(p['We2'], 8)
        be2 = _padcols(p['be2'].reshape(1, EF), 8)

        ean, sth = _edge_transform(xs, xd, eap, W1e, b1e, m1, s1v,
                                   p['ge1'].reshape(1, H), p['bbe1'].reshape(1, H),
                                   We2, be2, Wn1, bn1)
        mh, shv = _bnvec(sth, NEDGES)

        agg, _ = _aggregate(block_ptr, xs, ean, colp2, Wn1, bn1, mh, shv,
                            p['gn1'].reshape(1, H), p['bbn1'].reshape(1, H),
                            p['Wn1b'], p['bn1b'].reshape(1, H), batch_pad)

        stn = _node_stats(xp, agg, Wn2, bn2)
        mn, snv = _bnvec(stn, NNODES)
        Wn2b = _padcols(p['Wn2b'], 8)
        bn2b = _padcols(p['bn2b'].reshape(1, NF), 8)

        xp, px = _node_transform(xp, agg, Wn2, bn2, mn, snv,
                                 p['gn2'].reshape(1, H), p['bbn2'].reshape(1, H),
                                 Wn2b, bn2b, batch_pad)

        if it == N_MP - 1:
            xs2 = jnp.take(xp, rowp, axis=0)
            xs2 = jnp.pad(xs2, ((0, EPAD - NEDGES), (0, 0)))
            stg = _edge_stats(xs2, jnp.zeros_like(xs2), ean,
                              jnp.concatenate([Wg1[:8], jnp.zeros((8, H), jnp.float32),
                                               Wg1[8:]], axis=0), bg1)
            mg, sgv = _bnvec(stg, NEDGES)
            _, pg = _aggregate(block_ptr, xs2, ean, colp2, Wg1, bg1, mg, sgv,
                               p['gg1'].reshape(1, H), p['bbg1'].reshape(1, H),
                               p['Wg1b'], p['bg1b'].reshape(1, H), batch_pad)
            Wg2a = jnp.concatenate(
                [p['Wg2a'][:NF], jnp.zeros((1, H), jnp.float32), p['Wg2a'][NF:]],
                axis=0)
            u = _global_mlp(
                px, pg,
                Wg2a, p['bg2a'].reshape(1, H), p['gg2a'].reshape(1, H),
                p['bbg2a'].reshape(1, H),
                p['Wg2b'], p['bg2b'].reshape(1, H), p['gg2b'].reshape(1, H),
                p['bbg2b'].reshape(1, H),
                p['Wg2c'], p['bg2c'].reshape(1, H), p['gg2c'].reshape(1, H),
                p['bbg2c'].reshape(1, H),
                p['Wg2d'], p['bg2d'].reshape(1, GF))[:NGRAPH]
        eap = ean

    x_out = xp[:NNODES, :NF]
    ea_out = jnp.zeros((NEDGES, EF), jnp.float32).at[perm].set(eap[:NEDGES, :EF])
    return (x_out, ea_out, u)
